# Initial kernel scaffold; baseline (speedup 1.0000x reference)
import jax
import jax.numpy as jnp
from jax.experimental import pallas as pl


def kernel(x, codebook):
    raise NotImplementedError("write your pallas kernel here")


# speedup vs baseline: 1.0081x; 1.0081x over previous
import functools

import jax
import jax.numpy as jnp
from jax.experimental import pallas as pl
from jax.experimental.pallas import tpu as pltpu
from jax.experimental.pallas import tpu_sc as plsc

_TB = 256


def _dist_argmin_body(xsq_ref, xt_ref, cb_ref, idx_ref, loss_ref):
    K = cb_ref.shape[0]
    x_blk = xt_ref[...]
    cb = cb_ref[...]
    cross = jax.lax.dot_general(
        cb, x_blk, (((1,), (0,)), ((), ())),
        preferred_element_type=jnp.float32)
    c_sq = jnp.sum(cb * cb, axis=1, keepdims=True)
    s = xsq_ref[...] + c_sq
    dist = jnp.sqrt(jnp.maximum(s - 2.0 * cross, 0.0))
    m = jnp.min(dist, axis=0, keepdims=True)
    kio = jax.lax.broadcasted_iota(jnp.int32, dist.shape, 0)
    idx = jnp.min(jnp.where(dist == m, kio, K), axis=0, keepdims=True)
    idx_ref[...] = idx.reshape(1, 1, idx.shape[1])
    loss_ref[0, 0, 0] = jnp.sum(m * m)


def _distance_argmin(xsq_row, xt, codebook):
    C, N = xt.shape
    K = codebook.shape[0]
    grid = N // _TB
    idx3, loss_parts = pl.pallas_call(
        _dist_argmin_body,
        grid=(grid,),
        in_specs=[
            pl.BlockSpec((1, _TB), lambda i: (0, i)),
            pl.BlockSpec((C, _TB), lambda i: (0, i)),
            pl.BlockSpec((K, C), lambda i: (0, 0)),
        ],
        out_specs=[
            pl.BlockSpec((1, 1, _TB), lambda i: (i, 0, 0)),
            pl.BlockSpec((1, 1, 1), lambda i: (i, 0, 0), memory_space=pltpu.SMEM),
        ],
        out_shape=[
            jax.ShapeDtypeStruct((grid, 1, _TB), jnp.int32),
            jax.ShapeDtypeStruct((grid, 1, 1), jnp.float32),
        ],
    )(xsq_row, xt, codebook)
    return idx3.reshape(N), loss_parts


def _sc_gather(table, idx):
    V, D = table.shape
    B = idx.shape[0]
    info = plsc.get_sparse_core_info()
    nw = info.num_cores * info.num_subcores
    bpw = B // nw
    n_chunks = bpw // 128
    mesh = plsc.VectorSubcoreMesh(core_axis_name="c", subcore_axis_name="s")

    @functools.partial(
        pl.kernel, mesh=mesh,
        out_type=jax.ShapeDtypeStruct((B, D), jnp.float32),
        scratch_types=[
            pltpu.VMEM((bpw,), jnp.int32),
            pltpu.VMEM((bpw, D), jnp.float32),
            pltpu.SemaphoreType.DMA,
        ],
    )
    def g(table_hbm, idx_hbm, out_hbm, idx_v, rows_v, sem):
        wid = jax.lax.axis_index("s") * info.num_cores + jax.lax.axis_index("c")
        base = wid * bpw
        pltpu.sync_copy(idx_hbm.at[pl.ds(base, bpw)], idx_v)
        cps = [
            pltpu.async_copy(
                table_hbm.at[idx_v.at[pl.ds(j * 128, 128)]],
                rows_v.at[pl.ds(j * 128, 128)], sem)
            for j in range(n_chunks)
        ]
        for cp in cps:
            cp.wait()
        pltpu.sync_copy(rows_v, out_hbm.at[pl.ds(base, bpw)])

    return g(table, idx)


def kernel(x, codebook):
    B, C, H, W = x.shape
    N = B * H * W
    xp = jnp.transpose(x, (0, 2, 3, 1)).reshape(B, H * W, C)
    x_sq = jnp.sum(xp ** 2, axis=-1)
    xt = xp.reshape(N, C).T

    indices, loss_parts = _distance_argmin(x_sq.reshape(1, N), xt, codebook)
    cb_pad = jnp.pad(codebook, ((0, 0), (0, 128 - C)))
    quant = _sc_gather(cb_pad, indices)[:, :C]

    loss = jnp.sum(loss_parts) / (N * C)
    quant_out = jnp.transpose(quant.reshape(B, H, W, C), (0, 3, 1, 2))
    min_encoding_indices = indices.reshape(B, H, W)
    return (quant_out, loss, loss, min_encoding_indices)


# speedup vs baseline: 1.4730x; 1.4611x over previous
import functools

import jax
import jax.numpy as jnp
from jax.experimental import pallas as pl
from jax.experimental.pallas import tpu as pltpu
from jax.experimental.pallas import tpu_sc as plsc

_TB = 256


def _dist_argmin_body(xsq_ref, xt_ref, cb_ref, idx_ref, loss_ref):
    K = cb_ref.shape[0]
    x_blk = xt_ref[...]
    cb = cb_ref[...]
    cross = jax.lax.dot_general(
        cb, x_blk, (((1,), (0,)), ((), ())),
        preferred_element_type=jnp.float32)
    c_sq = jnp.sum(cb * cb, axis=1, keepdims=True)
    s = xsq_ref[...] + c_sq
    d2 = jnp.maximum(s - 2.0 * cross, 0.0)
    m2 = jnp.min(d2, axis=0, keepdims=True)
    m = jnp.sqrt(m2)
    t = m * m
    tb = jax.lax.bitcast_convert_type(t, jnp.int32)
    u = m2
    for off in (-3, -2, -1, 0, 1, 2, 3):
        cand = jax.lax.bitcast_convert_type(tb + off, jnp.float32)
        ok = (jnp.sqrt(cand) == m) & (cand > 0.0)
        u = jnp.where(ok, jnp.maximum(u, cand), u)
    kio = jax.lax.broadcasted_iota(jnp.int32, d2.shape, 0)
    idx = jnp.min(jnp.where(d2 <= u, kio, K), axis=0, keepdims=True)
    idx_ref[...] = idx.reshape(1, 1, idx.shape[1])
    loss_ref[0, 0, 0] = jnp.sum(m2)


def _distance_argmin(xsq_row, xt, codebook):
    C, N = xt.shape
    K = codebook.shape[0]
    grid = N // _TB
    idx3, loss_parts = pl.pallas_call(
        _dist_argmin_body,
        grid=(grid,),
        in_specs=[
            pl.BlockSpec((1, _TB), lambda i: (0, i)),
            pl.BlockSpec((C, _TB), lambda i: (0, i)),
            pl.BlockSpec((K, C), lambda i: (0, 0)),
        ],
        out_specs=[
            pl.BlockSpec((1, 1, _TB), lambda i: (i, 0, 0)),
            pl.BlockSpec((1, 1, 1), lambda i: (i, 0, 0), memory_space=pltpu.SMEM),
        ],
        out_shape=[
            jax.ShapeDtypeStruct((grid, 1, _TB), jnp.int32),
            jax.ShapeDtypeStruct((grid, 1, 1), jnp.float32),
        ],
    )(xsq_row, xt, codebook)
    return idx3.reshape(N), loss_parts


def _sc_gather(table, idx):
    V, D = table.shape
    B = idx.shape[0]
    info = plsc.get_sparse_core_info()
    nw = info.num_cores * info.num_subcores
    bpw = B // nw
    n_chunks = bpw // 128
    mesh = plsc.VectorSubcoreMesh(core_axis_name="c", subcore_axis_name="s")

    @functools.partial(
        pl.kernel, mesh=mesh,
        out_type=jax.ShapeDtypeStruct((B, D), jnp.float32),
        scratch_types=[
            pltpu.VMEM((bpw,), jnp.int32),
            pltpu.VMEM((bpw, D), jnp.float32),
            pltpu.SemaphoreType.DMA,
        ],
    )
    def g(table_hbm, idx_hbm, out_hbm, idx_v, rows_v, sem):
        wid = jax.lax.axis_index("s") * info.num_cores + jax.lax.axis_index("c")
        base = wid * bpw
        pltpu.sync_copy(idx_hbm.at[pl.ds(base, bpw)], idx_v)
        cps = [
            pltpu.async_copy(
                table_hbm.at[idx_v.at[pl.ds(j * 128, 128)]],
                rows_v.at[pl.ds(j * 128, 128)], sem)
            for j in range(n_chunks)
        ]
        for cp in cps:
            cp.wait()
        pltpu.sync_copy(rows_v, out_hbm.at[pl.ds(base, bpw)])

    return g(table, idx)


def kernel(x, codebook):
    B, C, H, W = x.shape
    N = B * H * W
    xp = jnp.transpose(x, (0, 2, 3, 1)).reshape(B, H * W, C)
    x_sq = jnp.sum(xp ** 2, axis=-1)
    xt = xp.reshape(N, C).T

    indices, loss_parts = _distance_argmin(x_sq.reshape(1, N), xt, codebook)
    cb_pad = jnp.pad(codebook, ((0, 0), (0, 128 - C)))
    quant = _sc_gather(cb_pad, indices)[:, :C]

    loss = jnp.sum(loss_parts) / (N * C)
    quant_out = jnp.transpose(quant.reshape(B, H, W, C), (0, 3, 1, 2))
    min_encoding_indices = indices.reshape(B, H, W)
    return (quant_out, loss, loss, min_encoding_indices)
